# Initial kernel scaffold; baseline (speedup 1.0000x reference)
import jax
import jax.numpy as jnp
from jax.experimental import pallas as pl


def kernel(x, W1, gamma, beta, running_mean, running_var, W2, b2):
    raise NotImplementedError("write your pallas kernel here")


# speedup vs baseline: 1.4407x; 1.4407x over previous
import jax
import jax.numpy as jnp
from jax.experimental import pallas as pl

_B, _CIN, _SY, _SX = 8, 96, 32, 32
_HID = 1024
_OUT = 425
_PIX = _SY * _SX
_EPS = 1e-5


def _head_kernel(x_ref, w1_ref, g_ref, b_ref, m_ref, v_ref, w2_ref, b2_ref,
                 o_ref):
    acc = jnp.zeros((_PIX, _HID), dtype=jnp.float32)
    for dy in range(3):
        for dx in range(3):
            k = dy * 3 + dx
            patch = x_ref[0, dy:dy + _SY, dx:dx + _SX, :].reshape(_PIX, _CIN)
            acc += jnp.dot(patch, w1_ref[k],
                           preferred_element_type=jnp.float32)
    scale = g_ref[...] * jax.lax.rsqrt(v_ref[...] + _EPS)
    shift = b_ref[...] - m_ref[...] * scale
    h = acc * scale + shift
    h = jnp.where(h >= 0, h, 0.1 * h)
    out = jnp.dot(h.astype(jnp.bfloat16), w2_ref[...],
                  preferred_element_type=jnp.float32) + b2_ref[...]
    o_ref[...] = out.reshape(1, _SY, _SX, _OUT)


def kernel(x, W1, gamma, beta, running_mean, running_var, W2, b2):
    xp = jnp.transpose(x, (0, 2, 3, 1))
    xp = jnp.pad(xp, ((0, 0), (1, 1), (1, 1), (0, 0))).astype(jnp.bfloat16)
    w1 = jnp.transpose(W1, (2, 3, 1, 0)).reshape(9, _CIN, _HID)
    w1 = w1.astype(jnp.bfloat16)
    w2 = jnp.transpose(W2.reshape(_OUT, _HID)).astype(jnp.bfloat16)

    out = pl.pallas_call(
        _head_kernel,
        grid=(_B,),
        in_specs=[
            pl.BlockSpec((1, _SY + 2, _SX + 2, _CIN), lambda b: (b, 0, 0, 0)),
            pl.BlockSpec((9, _CIN, _HID), lambda b: (0, 0, 0)),
            pl.BlockSpec((1, _HID), lambda b: (0, 0)),
            pl.BlockSpec((1, _HID), lambda b: (0, 0)),
            pl.BlockSpec((1, _HID), lambda b: (0, 0)),
            pl.BlockSpec((1, _HID), lambda b: (0, 0)),
            pl.BlockSpec((_HID, _OUT), lambda b: (0, 0)),
            pl.BlockSpec((1, _OUT), lambda b: (0, 0)),
        ],
        out_specs=pl.BlockSpec((1, _SY, _SX, _OUT), lambda b: (b, 0, 0, 0)),
        out_shape=jax.ShapeDtypeStruct((_B, _SY, _SX, _OUT), jnp.float32),
    )(xp, w1,
      gamma.reshape(1, _HID), beta.reshape(1, _HID),
      running_mean.reshape(1, _HID), running_var.reshape(1, _HID),
      w2, b2.reshape(1, _OUT))
    return out


# speedup vs baseline: 1.8053x; 1.2531x over previous
import jax
import jax.numpy as jnp
from jax.experimental import pallas as pl

_B, _CIN, _SY, _SX = 8, 96, 32, 32
_CPAD = 128
_HID = 1024
_OUT = 425
_PIX = _SY * _SX
_EPS = 1e-5


def _head_kernel(x_ref, w1_ref, g_ref, b_ref, m_ref, v_ref, w2_ref, b2_ref,
                 o_ref):
    pieces = []
    for dy in range(3):
        for dx in range(3):
            patch = x_ref[0, dx, dy:dy + _SY, :, :].reshape(_PIX, _CIN)
            pieces.append(jnp.pad(patch, ((0, 0), (0, _CPAD - _CIN))))
    col = jnp.concatenate(pieces, axis=1)
    acc = jnp.dot(col, w1_ref[...], preferred_element_type=jnp.float32)
    scale = g_ref[...] * jax.lax.rsqrt(v_ref[...] + _EPS)
    shift = b_ref[...] - m_ref[...] * scale
    h = acc * scale + shift
    h = jnp.where(h >= 0, h, 0.1 * h)
    out = jnp.dot(h.astype(jnp.bfloat16), w2_ref[...],
                  preferred_element_type=jnp.float32) + b2_ref[...]
    o_ref[...] = out.reshape(1, _SY, _SX, _OUT)


def kernel(x, W1, gamma, beta, running_mean, running_var, W2, b2):
    xp = jnp.transpose(x, (0, 2, 3, 1))
    xp = jnp.pad(xp, ((0, 0), (1, 1), (1, 1), (0, 0))).astype(jnp.bfloat16)
    xs = jnp.stack([xp[:, :, dx:dx + _SX, :] for dx in range(3)], axis=1)

    w1 = jnp.transpose(W1, (2, 3, 1, 0)).reshape(9, _CIN, _HID)
    w1 = jnp.pad(w1, ((0, 0), (0, _CPAD - _CIN), (0, 0)))
    w1 = w1.reshape(9 * _CPAD, _HID).astype(jnp.bfloat16)
    w2 = jnp.transpose(W2.reshape(_OUT, _HID)).astype(jnp.bfloat16)

    out = pl.pallas_call(
        _head_kernel,
        grid=(_B,),
        in_specs=[
            pl.BlockSpec((1, 3, _SY + 2, _SX, _CIN),
                         lambda b: (b, 0, 0, 0, 0)),
            pl.BlockSpec((9 * _CPAD, _HID), lambda b: (0, 0)),
            pl.BlockSpec((1, _HID), lambda b: (0, 0)),
            pl.BlockSpec((1, _HID), lambda b: (0, 0)),
            pl.BlockSpec((1, _HID), lambda b: (0, 0)),
            pl.BlockSpec((1, _HID), lambda b: (0, 0)),
            pl.BlockSpec((_HID, _OUT), lambda b: (0, 0)),
            pl.BlockSpec((1, _OUT), lambda b: (0, 0)),
        ],
        out_specs=pl.BlockSpec((1, _SY, _SX, _OUT), lambda b: (b, 0, 0, 0)),
        out_shape=jax.ShapeDtypeStruct((_B, _SY, _SX, _OUT), jnp.float32),
    )(xs, w1,
      gamma.reshape(1, _HID), beta.reshape(1, _HID),
      running_mean.reshape(1, _HID), running_var.reshape(1, _HID),
      w2, b2.reshape(1, _OUT))
    return out
